# baseline (speedup 1.0000x reference)
import functools

import jax
import jax.numpy as jnp
from jax.experimental import pallas as pl
from jax.experimental.pallas import tpu as pltpu

_G = 32
_S = 2


def _sliding_max(m, window_size, L):
    span = 1
    while span * 2 <= window_size:
        m = jnp.maximum(m, pltpu.roll(m, L - span, 1))
        span *= 2
    if span < window_size:
        s = window_size - span
        m = jnp.maximum(m, pltpu.roll(m, L - s, 1))
    return m


def _whl_body(x_ref, a1_ref, b1_ref, a2_ref, b2_ref, a3_ref, b3_ref,
              out_ref, *, window_size, n_w, C, C2, C4, L, n_chunks):
    R = _G * C
    f32 = jnp.float32
    neg = jnp.finfo(f32).min
    a1 = a1_ref[...]
    a2 = a2_ref[...]
    a3 = a3_ref[...]
    b1 = b1_ref[...]
    b2 = b2_ref[...]
    def trees(k0):
        ks = range(k0, min(k0 + _S, n_chunks))
        return [_sliding_max(x_ref[k * R:(k + 1) * R, :].astype(f32),
                             window_size, L) for k in ks]

    ms = trees(0)
    for k0 in range(0, n_chunks, _S):
        ks = range(k0, min(k0 + _S, n_chunks))
        hs = [jnp.dot(a1, m, preferred_element_type=f32) + b1 for m in ms]
        ms = trees(k0 + _S) if k0 + _S < n_chunks else None
        hs = [h * jax.nn.sigmoid(h) for h in hs]
        gs = [jnp.dot(a2, h, preferred_element_type=f32) + b2 for h in hs]
        gs = [g * jax.nn.sigmoid(g) for g in gs]
        ls = [jnp.dot(a3, g, preferred_element_type=f32) + b3_ref[0]
              for g in gs]
        for k, logits in zip(ks, ls):
            col = jax.lax.broadcasted_iota(jnp.int32, logits.shape, 1)
            res = jnp.max(jnp.where(col < n_w, logits, neg),
                          axis=1, keepdims=True)
            out_ref[k * _G:(k + 1) * _G, :] = res


def kernel(x, w1, b1, w2, b2, w3, b3):
    window_size = 16
    B, C, L = x.shape
    C2, C4 = w1.shape[0], w2.shape[0]
    n_w = L - window_size + 1

    itemsize = int(jnp.dtype(x.dtype).itemsize)
    Bt = int(max(_G, min(B, (4 << 20) // max(1, C * L * itemsize))))
    Bt -= Bt % _G
    n_blocks = pl.cdiv(B, Bt)
    Bpad = n_blocks * Bt
    x_in = x
    if Bpad != B:
        x_in = jnp.pad(x, ((0, Bpad - B), (0, 0), (0, 0)))
    xf = x_in.reshape(Bpad * C, L)
    n_chunks = Bt // _G

    f32 = jnp.float32
    eye = jnp.eye(_G, dtype=f32)
    a1 = jnp.kron(eye, jnp.asarray(w1, f32))
    a2 = jnp.kron(eye, jnp.asarray(w2, f32))
    a3 = jnp.kron(eye, jnp.asarray(w3, f32)[None, :])
    b1t = jnp.tile(jnp.asarray(b1, f32), _G)[:, None]
    b2t = jnp.tile(jnp.asarray(b2, f32), _G)[:, None]

    smem = pl.BlockSpec(memory_space=pltpu.MemorySpace.SMEM)
    body = functools.partial(_whl_body, window_size=window_size, n_w=n_w,
                             C=C, C2=C2, C4=C4, L=L, n_chunks=n_chunks)

    out = pl.pallas_call(
        body,
        out_shape=jax.ShapeDtypeStruct((Bpad, 1), f32),
        grid=(n_blocks,),
        in_specs=[
            pl.BlockSpec((Bt * C, L), lambda b: (b, 0)),
            pl.BlockSpec((_G * C2, _G * C), lambda b: (0, 0)),
            pl.BlockSpec((_G * C2, 1), lambda b: (0, 0)),
            pl.BlockSpec((_G * C4, _G * C2), lambda b: (0, 0)),
            pl.BlockSpec((_G * C4, 1), lambda b: (0, 0)),
            pl.BlockSpec((_G, _G * C4), lambda b: (0, 0)),
            smem,
        ],
        out_specs=pl.BlockSpec((Bt, 1), lambda b: (b, 0)),
        compiler_params=pltpu.CompilerParams(
            dimension_semantics=("arbitrary",),
            vmem_limit_bytes=64 * 1024 * 1024),
        cost_estimate=pl.CostEstimate(
            flops=2 * B * n_w * (C * C2 + C2 * C4 + C4),
            transcendentals=B * n_w * (C2 + C4),
            bytes_accessed=B * C * L * itemsize + B * 4),
    )(xf, a1, b1t, a2, b2t, a3, jnp.asarray(b3, f32))

    return out[:B]


# speedup vs baseline: 1.0119x; 1.0119x over previous
import functools

import jax
import jax.numpy as jnp
from jax.experimental import pallas as pl
from jax.experimental.pallas import tpu as pltpu

_G = 32
_S = 2


def _sliding_max(m, window_size, L):
    if L % 128 == 0 and window_size <= 128:
        return _sliding_max_cols(m, window_size, L)
    span = 1
    while span * 2 <= window_size:
        m = jnp.maximum(m, pltpu.roll(m, L - span, 1))
        span *= 2
    if span < window_size:
        s = window_size - span
        m = jnp.maximum(m, pltpu.roll(m, L - s, 1))
    return m


def _sliding_max_cols(m, window_size, L):
    nv = L // 128
    vs = [m[:, j * 128:(j + 1) * 128] for j in range(nv)]
    lane = jax.lax.broadcasted_iota(jnp.int32, vs[0].shape, 1)
    span = 1
    steps = []
    while span * 2 <= window_size:
        steps.append(span)
        span *= 2
    if span < window_size:
        steps.append(window_size - span)
    for s in steps:
        rs = [pltpu.roll(v, 128 - s, 1) for v in vs]
        sel = lane >= 128 - s
        vs = [jnp.maximum(vs[j], jnp.where(sel, rs[j + 1], rs[j]))
              for j in range(nv - 1)] + [jnp.maximum(vs[-1], rs[-1])]
    return jnp.concatenate(vs, axis=1)


def _whl_body(x_ref, a1_ref, b1_ref, a2_ref, b2_ref, a3_ref, b3_ref,
              out_ref, *, window_size, n_w, C, C2, C4, L, n_chunks):
    R = _G * C
    f32 = jnp.float32
    neg = jnp.finfo(f32).min
    a1 = a1_ref[...]
    a2 = a2_ref[...]
    a3 = a3_ref[...]
    b1 = b1_ref[...]
    b2 = b2_ref[...]
    def trees(k0):
        ks = range(k0, min(k0 + _S, n_chunks))
        return [_sliding_max(x_ref[k * R:(k + 1) * R, :].astype(f32),
                             window_size, L) for k in ks]

    ms = trees(0)
    for k0 in range(0, n_chunks, _S):
        ks = range(k0, min(k0 + _S, n_chunks))
        hs = [jnp.dot(a1, m, preferred_element_type=f32) + b1 for m in ms]
        ms = trees(k0 + _S) if k0 + _S < n_chunks else None
        hs = [h * jax.lax.logistic(h) for h in hs]
        gs = [jnp.dot(a2, h, preferred_element_type=f32) + b2 for h in hs]
        gs = [g * jax.lax.logistic(g) for g in gs]
        ls = [jnp.dot(a3, g, preferred_element_type=f32) + b3_ref[0]
              for g in gs]
        for k, logits in zip(ks, ls):
            col = jax.lax.broadcasted_iota(jnp.int32, logits.shape, 1)
            res = jnp.max(jnp.where(col < n_w, logits, neg),
                          axis=1, keepdims=True)
            out_ref[k * _G:(k + 1) * _G, :] = res


def kernel(x, w1, b1, w2, b2, w3, b3):
    window_size = 16
    B, C, L = x.shape
    C2, C4 = w1.shape[0], w2.shape[0]
    n_w = L - window_size + 1

    itemsize = int(jnp.dtype(x.dtype).itemsize)
    Bt = int(max(_G, min(B, (4 << 20) // max(1, C * L * itemsize))))
    Bt -= Bt % _G
    n_blocks = pl.cdiv(B, Bt)
    Bpad = n_blocks * Bt
    x_in = x
    if Bpad != B:
        x_in = jnp.pad(x, ((0, Bpad - B), (0, 0), (0, 0)))
    xf = x_in.reshape(Bpad * C, L)
    n_chunks = Bt // _G

    f32 = jnp.float32
    eye = jnp.eye(_G, dtype=f32)
    a1 = jnp.kron(eye, jnp.asarray(w1, f32))
    a2 = jnp.kron(eye, jnp.asarray(w2, f32))
    a3 = jnp.kron(eye, jnp.asarray(w3, f32)[None, :])
    b1t = jnp.tile(jnp.asarray(b1, f32), _G)[:, None]
    b2t = jnp.tile(jnp.asarray(b2, f32), _G)[:, None]

    smem = pl.BlockSpec(memory_space=pltpu.MemorySpace.SMEM)
    body = functools.partial(_whl_body, window_size=window_size, n_w=n_w,
                             C=C, C2=C2, C4=C4, L=L, n_chunks=n_chunks)

    out = pl.pallas_call(
        body,
        out_shape=jax.ShapeDtypeStruct((Bpad, 1), f32),
        grid=(n_blocks,),
        in_specs=[
            pl.BlockSpec((Bt * C, L), lambda b: (b, 0)),
            pl.BlockSpec((_G * C2, _G * C), lambda b: (0, 0)),
            pl.BlockSpec((_G * C2, 1), lambda b: (0, 0)),
            pl.BlockSpec((_G * C4, _G * C2), lambda b: (0, 0)),
            pl.BlockSpec((_G * C4, 1), lambda b: (0, 0)),
            pl.BlockSpec((_G, _G * C4), lambda b: (0, 0)),
            smem,
        ],
        out_specs=pl.BlockSpec((Bt, 1), lambda b: (b, 0)),
        compiler_params=pltpu.CompilerParams(
            dimension_semantics=("arbitrary",),
            vmem_limit_bytes=64 * 1024 * 1024),
        cost_estimate=pl.CostEstimate(
            flops=2 * B * n_w * (C * C2 + C2 * C4 + C4),
            transcendentals=B * n_w * (C2 + C4),
            bytes_accessed=B * C * L * itemsize + B * 4),
    )(xf, a1, b1t, a2, b2t, a3, jnp.asarray(b3, f32))

    return out[:B]


# speedup vs baseline: 1.0234x; 1.0113x over previous
import functools

import jax
import jax.numpy as jnp
from jax.experimental import pallas as pl
from jax.experimental.pallas import tpu as pltpu

_G = 32
_S = 2


def _sliding_max(m, window_size, L):
    if L % 128 == 0 and window_size <= 128:
        return _sliding_max_cols(m, window_size, L)
    span = 1
    while span * 2 <= window_size:
        m = jnp.maximum(m, pltpu.roll(m, L - span, 1))
        span *= 2
    if span < window_size:
        s = window_size - span
        m = jnp.maximum(m, pltpu.roll(m, L - s, 1))
    return m


def _sliding_max_cols(m, window_size, L):
    nv = L // 128
    vs = [m[:, j * 128:(j + 1) * 128] for j in range(nv)]
    lane = jax.lax.broadcasted_iota(jnp.int32, vs[0].shape, 1)
    span = 1
    steps = []
    while span * 2 <= window_size:
        steps.append(span)
        span *= 2
    if span < window_size:
        steps.append(window_size - span)
    for s in steps:
        rs = [pltpu.roll(v, 128 - s, 1) for v in vs]
        sel = lane >= 128 - s
        vs = [jnp.maximum(vs[j], jnp.where(sel, rs[j + 1], rs[j]))
              for j in range(nv - 1)] + [jnp.maximum(vs[-1], rs[-1])]
    return jnp.concatenate(vs, axis=1)


def _whl_body(x_ref, a1_ref, b1_ref, a2_ref, b2_ref, a3_ref, b3_ref,
              out_ref, *, window_size, n_w, C, C2, C4, L, n_chunks):
    R = _G * C
    f32 = jnp.float32
    neg = jnp.finfo(f32).min
    a1 = a1_ref[...]
    a2 = a2_ref[...]
    a3 = a3_ref[...]
    b1 = b1_ref[...]
    b2 = b2_ref[...]
    def trees(k0):
        ks = range(k0, min(k0 + _S, n_chunks))
        return [_sliding_max(x_ref[k * R:(k + 1) * R, :].astype(f32),
                             window_size, L) for k in ks]

    ms = trees(0)
    for k0 in range(0, n_chunks, _S):
        ks = range(k0, min(k0 + _S, n_chunks))
        hs = [jnp.dot(a1, m, preferred_element_type=f32) + b1 for m in ms]
        ms = trees(k0 + _S) if k0 + _S < n_chunks else None
        hs = [h * jax.lax.logistic(h) for h in hs]
        gs = [jnp.dot(a2, h, preferred_element_type=f32) + b2 for h in hs]
        gs = [g * jax.lax.logistic(g) for g in gs]
        ls = [jnp.dot(a3, g, preferred_element_type=f32) + b3_ref[0]
              for g in gs]
        for k, logits in zip(ks, ls):
            col = jax.lax.broadcasted_iota(jnp.int32, logits.shape, 1)
            res = jnp.max(jnp.where(col < n_w, logits, neg),
                          axis=1, keepdims=True)
            out_ref[k * _G:(k + 1) * _G, :] = res


def kernel(x, w1, b1, w2, b2, w3, b3):
    window_size = 16
    B, C, L = x.shape
    C2, C4 = w1.shape[0], w2.shape[0]
    n_w = L - window_size + 1

    itemsize = int(jnp.dtype(x.dtype).itemsize)
    Bt = int(max(_G, min(B, (8 << 20) // max(1, C * L * itemsize))))
    Bt -= Bt % _G
    n_blocks = pl.cdiv(B, Bt)
    Bpad = n_blocks * Bt
    x_in = x
    if Bpad != B:
        x_in = jnp.pad(x, ((0, Bpad - B), (0, 0), (0, 0)))
    xf = x_in.reshape(Bpad * C, L)
    n_chunks = Bt // _G

    f32 = jnp.float32
    eye = jnp.eye(_G, dtype=f32)
    a1 = jnp.kron(eye, jnp.asarray(w1, f32))
    a2 = jnp.kron(eye, jnp.asarray(w2, f32))
    a3 = jnp.kron(eye, jnp.asarray(w3, f32)[None, :])
    b1t = jnp.tile(jnp.asarray(b1, f32), _G)[:, None]
    b2t = jnp.tile(jnp.asarray(b2, f32), _G)[:, None]

    smem = pl.BlockSpec(memory_space=pltpu.MemorySpace.SMEM)
    body = functools.partial(_whl_body, window_size=window_size, n_w=n_w,
                             C=C, C2=C2, C4=C4, L=L, n_chunks=n_chunks)

    out = pl.pallas_call(
        body,
        out_shape=jax.ShapeDtypeStruct((Bpad, 1), f32),
        grid=(n_blocks,),
        in_specs=[
            pl.BlockSpec((Bt * C, L), lambda b: (b, 0)),
            pl.BlockSpec((_G * C2, _G * C), lambda b: (0, 0)),
            pl.BlockSpec((_G * C2, 1), lambda b: (0, 0)),
            pl.BlockSpec((_G * C4, _G * C2), lambda b: (0, 0)),
            pl.BlockSpec((_G * C4, 1), lambda b: (0, 0)),
            pl.BlockSpec((_G, _G * C4), lambda b: (0, 0)),
            smem,
        ],
        out_specs=pl.BlockSpec((Bt, 1), lambda b: (b, 0)),
        compiler_params=pltpu.CompilerParams(
            dimension_semantics=("arbitrary",),
            vmem_limit_bytes=64 * 1024 * 1024),
        cost_estimate=pl.CostEstimate(
            flops=2 * B * n_w * (C * C2 + C2 * C4 + C4),
            transcendentals=B * n_w * (C2 + C4),
            bytes_accessed=B * C * L * itemsize + B * 4),
    )(xf, a1, b1t, a2, b2t, a3, jnp.asarray(b3, f32))

    return out[:B]
